# baseline (speedup 1.0000x reference)
import functools
import math

import jax
import jax.numpy as jnp
from jax import lax
from jax.experimental import pallas as pl
from jax.experimental.pallas import tpu as pltpu
from jax.experimental.pallas import tpu_sc as plsc

D_MODEL = 64
SCALE = math.sqrt(D_MODEL)
NC = 2
NS = 16
NW = NC * NS
LANES = 16


def _make_lookup(R, S, C, NB):
    rows_per_w = R // NW
    halves = S // C
    n_chunks = rows_per_w * halves
    assert rows_per_w * NW == R and halves * C == S and n_chunks >= 2 * NB

    mesh = plsc.VectorSubcoreMesh(core_axis_name="c", subcore_axis_name="s")

    scratch = [
        pltpu.VMEM((rows_per_w, S), jnp.int32),
        pltpu.VMEM((NB, C, D_MODEL), jnp.float32),
        pltpu.VMEM((NB, C, D_MODEL), jnp.float32),
    ] + [pltpu.SemaphoreType.DMA] * (2 * NB)

    @functools.partial(
        pl.kernel,
        mesh=mesh,
        out_type=jax.ShapeDtypeStruct((R, S, D_MODEL), jnp.float32),
        scratch_types=scratch,
        compiler_params=pltpu.CompilerParams(use_tc_tiling_on_sc=False),
    )
    def lookup(x_hbm, table_hbm, out_hbm, idx_v, gbuf, obuf, *sems):
        gsem = sems[:NB]
        osem = sems[NB:]
        wid = lax.axis_index("s") * NC + lax.axis_index("c")
        base_row = wid * rows_per_w

        pltpu.sync_copy(x_hbm.at[pl.ds(base_row, rows_per_w)], idx_v)

        def fire_gather(g, b):
            r = g // halves
            h = g % halves
            idx = idx_v.at[r, pl.ds(h * C, C)]
            pltpu.async_copy(table_hbm.at[idx], gbuf.at[b], gsem[b])

        def wait_gather(b):
            pltpu.make_async_copy(
                table_hbm.at[pl.ds(0, C)], gbuf.at[b], gsem[b]).wait()

        def fire_write(g, b):
            r = g // halves
            h = g % halves
            dst = out_hbm.at[base_row + r, pl.ds(h * C, C)]
            pltpu.async_copy(obuf.at[b], dst, osem[b])

        def wait_write(b):
            pltpu.make_async_copy(
                obuf.at[b], out_hbm.at[0, pl.ds(0, C)], osem[b]).wait()

        def scale(b):
            src = gbuf.at[b]
            dst = obuf.at[b]

            @pl.loop(0, C, unroll=4)
            def _(i):
                for j in range(D_MODEL // LANES):
                    s = pl.ds(j * LANES, LANES)
                    dst[i, s] = src[i, s] * SCALE

        for b in range(NB):
            fire_gather(b, b)

        for b in range(NB):
            wait_gather(b)
            scale(b)
            fire_gather(b + NB, b)
            fire_write(b, b)

        @pl.loop(NB, n_chunks - NB, step=NB)
        def _(G):
            for b in range(NB):
                g = G + b
                wait_gather(b)
                wait_write(b)
                scale(b)
                fire_gather(g + NB, b)
                fire_write(g, b)

        for b in range(NB):
            wait_gather(b)
            wait_write(b)
            scale(b)
            fire_write(n_chunks - NB + b, b)
        for b in range(NB):
            wait_write(b)

    return lookup


def kernel(x, table):
    R, S = x.shape
    idx = x.astype(jnp.int32)
    return _make_lookup(R, S, S, 4)(idx, table)


# speedup vs baseline: 1.0757x; 1.0757x over previous
import functools
import math

import jax
import jax.numpy as jnp
from jax import lax
from jax.experimental import pallas as pl
from jax.experimental.pallas import tpu as pltpu
from jax.experimental.pallas import tpu_sc as plsc

D_MODEL = 64
SCALE = math.sqrt(D_MODEL)
NC = 2
NS = 16
NW = NC * NS
LANES = 16


def _make_lookup(R, S, C, NB):
    rows_per_w = R // NW
    splits = S // C
    n_chunks = rows_per_w * splits
    assert rows_per_w * NW == R and splits * C == S and n_chunks >= 2 * NB
    assert C % 8 == 0

    mesh = plsc.VectorSubcoreMesh(core_axis_name="c", subcore_axis_name="s")

    scratch = [
        pltpu.VMEM((rows_per_w, S), jnp.int32),
        pltpu.VMEM((NB, C, D_MODEL), jnp.float32),
        pltpu.VMEM((NB, C, 128), jnp.float32),
    ] + [pltpu.SemaphoreType.DMA] * (2 * NB)

    @functools.partial(
        pl.kernel,
        mesh=mesh,
        out_type=jax.ShapeDtypeStruct((R, S, 128), jnp.float32),
        scratch_types=scratch,
        compiler_params=pltpu.CompilerParams(use_tc_tiling_on_sc=False),
    )
    def lookup(x_hbm, table_hbm, out_hbm, idx_v, gbuf, obuf, *sems):
        gsem = sems[:NB]
        osem = sems[NB:]
        wid = lax.axis_index("s") * NC + lax.axis_index("c")
        base_row = wid * rows_per_w

        pltpu.sync_copy(x_hbm.at[pl.ds(base_row, rows_per_w)], idx_v)

        def fire_gather(g, b):
            r = g // splits
            h = g % splits
            start = pl.multiple_of(h * C, 8)
            idx = idx_v.at[r, pl.ds(start, C)]
            pltpu.async_copy(table_hbm.at[idx], gbuf.at[b], gsem[b])

        def wait_gather(b):
            pltpu.make_async_copy(
                table_hbm.at[pl.ds(0, C)], gbuf.at[b], gsem[b]).wait()

        def fire_write(g, b):
            r = g // splits
            h = g % splits
            dst = out_hbm.at[base_row + r, pl.ds(pl.multiple_of(h * C, 8), C)]
            pltpu.async_copy(obuf.at[b], dst, osem[b])

        def wait_write(b):
            pltpu.make_async_copy(
                obuf.at[b], out_hbm.at[0, pl.ds(0, C)], osem[b]).wait()

        def scale(b):
            src = gbuf.at[b]
            dst = obuf.at[b]

            @pl.loop(0, C, unroll=4)
            def _(i):
                for j in range(D_MODEL // LANES):
                    s = pl.ds(j * LANES, LANES)
                    dst[i, s] = src[i, s] * SCALE

        for b in range(NB):
            fire_gather(b, b)

        for b in range(NB):
            wait_gather(b)
            scale(b)
            fire_gather(b + NB, b)
            fire_write(b, b)

        @pl.loop(NB, n_chunks - NB, step=NB)
        def _(G):
            for b in range(NB):
                g = G + b
                wait_gather(b)
                wait_write(b)
                scale(b)
                fire_gather(g + NB, b)
                fire_write(g, b)

        for b in range(NB):
            wait_gather(b)
            wait_write(b)
            scale(b)
            fire_write(n_chunks - NB + b, b)
        for b in range(NB):
            wait_write(b)

    return lookup


def kernel(x, table):
    R, S = x.shape
    idx = x.astype(jnp.int32)
    out = _make_lookup(R, S, 40, 4)(idx, table)
    return out[:, :, :D_MODEL]


# speedup vs baseline: 1.1701x; 1.0878x over previous
import functools
import math

import jax
import jax.numpy as jnp
from jax import lax
from jax.experimental import pallas as pl
from jax.experimental.pallas import tpu as pltpu
from jax.experimental.pallas import tpu_sc as plsc

D_MODEL = 64
SCALE = math.sqrt(D_MODEL)
NC = 2
NS = 16
NW = NC * NS


def _make_lookup(R, S, C, NB, K):
    rows_per_w = R // NW
    splits = S // C
    n_chunks = rows_per_w * splits
    assert rows_per_w * NW == R and splits * C == S
    assert C % 8 == 0 and 0 < K < NB and n_chunks % NB == 0
    assert n_chunks > 2 * NB

    mesh = plsc.VectorSubcoreMesh(core_axis_name="c", subcore_axis_name="s")

    scratch = [
        pltpu.VMEM((rows_per_w, S), jnp.int32),
        pltpu.VMEM((NB, C, 128), jnp.float32),
    ] + [pltpu.SemaphoreType.DMA] * (2 * NB)

    @functools.partial(
        pl.kernel,
        mesh=mesh,
        out_type=jax.ShapeDtypeStruct((R, S, 128), jnp.float32),
        scratch_types=scratch,
        compiler_params=pltpu.CompilerParams(use_tc_tiling_on_sc=False),
    )
    def lookup(x_hbm, table_hbm, out_hbm, idx_v, ring, *sems):
        gsem = sems[:NB]
        osem = sems[NB:]
        wid = lax.axis_index("s") * NC + lax.axis_index("c")
        base_row = wid * rows_per_w

        pltpu.sync_copy(x_hbm.at[pl.ds(base_row, rows_per_w)], idx_v)

        def fire_gather(g, b):
            r = g // splits
            h = g % splits
            start = pl.multiple_of(h * C, 8)
            idx = idx_v.at[r, pl.ds(start, C)]
            pltpu.async_copy(table_hbm.at[idx], ring.at[b], gsem[b])

        def wait_gather(b):
            pltpu.make_async_copy(
                table_hbm.at[pl.ds(0, C)], ring.at[b], gsem[b]).wait()

        def fire_write(g, b):
            r = g // splits
            h = g % splits
            dst = out_hbm.at[base_row + r, pl.ds(pl.multiple_of(h * C, 8), C)]
            pltpu.async_copy(ring.at[b], dst, osem[b])

        def wait_write(b):
            pltpu.make_async_copy(
                ring.at[b], out_hbm.at[0, pl.ds(0, C)], osem[b]).wait()

        for f in range(K):
            fire_gather(f, f)

        for g in range(NB - K):
            wait_gather(g % NB)
            fire_write(g, g % NB)
            fire_gather(g + K, (g + K) % NB)

        @pl.loop(NB - K, n_chunks - K, step=NB)
        def _(G):
            for db in range(NB):
                g = G + db
                b = (NB - K + db) % NB
                wait_gather(b)
                fire_write(g, b)
                bf = (b + K) % NB
                wait_write(bf)
                fire_gather(g + K, bf)

        for dg in range(K):
            b = (NB - K + dg) % NB
            wait_gather(b)
            fire_write(n_chunks - K + dg, b)
        for b in range(NB):
            wait_write(b)

    return lookup


def kernel(x, table):
    R, S = x.shape
    idx = x.astype(jnp.int32)
    staged = jnp.concatenate([table, table], axis=1) * jnp.float32(SCALE)
    out = _make_lookup(R, S, 40, 8, 4)(idx, staged)
    return out[:, :, :D_MODEL]


# speedup vs baseline: 1.4153x; 1.2095x over previous
import functools
import math

import jax
import jax.numpy as jnp
from jax import lax
from jax.experimental import pallas as pl
from jax.experimental.pallas import tpu as pltpu
from jax.experimental.pallas import tpu_sc as plsc

D_MODEL = 64
SCALE = math.sqrt(D_MODEL)
NC = 2
NS = 16
NW = NC * NS


def _make_lookup(R, S, C, NB, K):
    rows_per_w = R // NW
    splits = S // C
    n_chunks = rows_per_w * splits
    assert rows_per_w * NW == R and splits * C == S
    assert C % 8 == 0 and 0 < K < NB and n_chunks % NB == 0
    assert n_chunks > 2 * NB

    mesh = plsc.VectorSubcoreMesh(core_axis_name="c", subcore_axis_name="s")

    scratch = [
        pltpu.VMEM((rows_per_w, S), jnp.int32),
        pltpu.VMEM((NB, C, D_MODEL), jnp.float32),
    ] + [pltpu.SemaphoreType.DMA] * (2 * NB)

    @functools.partial(
        pl.kernel,
        mesh=mesh,
        out_type=jax.ShapeDtypeStruct((R, S, 128), jnp.float32),
        scratch_types=scratch,
        compiler_params=pltpu.CompilerParams(use_tc_tiling_on_sc=False),
    )
    def lookup(x_hbm, table_hbm, out_hbm, idx_v, ring, *sems):
        gsem = sems[:NB]
        osem = sems[NB:]
        wid = lax.axis_index("s") * NC + lax.axis_index("c")
        base_row = wid * rows_per_w

        pltpu.sync_copy(x_hbm.at[pl.ds(base_row, rows_per_w)], idx_v)

        def fire_gather(g, b):
            r = g // splits
            h = g % splits
            start = pl.multiple_of(h * C, 8)
            idx = idx_v.at[r, pl.ds(start, C)]
            pltpu.async_copy(table_hbm.at[idx], ring.at[b], gsem[b])

        def wait_gather(b):
            pltpu.make_async_copy(
                table_hbm.at[pl.ds(0, C)], ring.at[b], gsem[b]).wait()

        def fire_write(g, b):
            r = g // splits
            h = g % splits
            dst = out_hbm.at[base_row + r, pl.ds(pl.multiple_of(h * C, 8), C),
                             pl.ds(0, D_MODEL)]
            pltpu.async_copy(ring.at[b], dst, osem[b])

        def wait_write(b):
            pltpu.make_async_copy(
                ring.at[b], out_hbm.at[0, pl.ds(0, C), pl.ds(0, D_MODEL)],
                osem[b]).wait()

        def scale(b):
            dst = ring.at[b]

            @pl.loop(0, C, unroll=4)
            def _(i):
                for j in range(D_MODEL // 16):
                    s = pl.ds(j * 16, 16)
                    dst[i, s] = dst[i, s] * SCALE

        for f in range(K):
            fire_gather(f, f)

        for g in range(NB - K):
            wait_gather(g % NB)
            scale(g % NB)
            fire_write(g, g % NB)
            fire_gather(g + K, (g + K) % NB)

        @pl.loop(NB - K, n_chunks - K, step=NB)
        def _(G):
            for db in range(NB):
                g = G + db
                b = (NB - K + db) % NB
                wait_gather(b)
                scale(b)
                fire_write(g, b)
                bf = (b + K) % NB
                wait_write(bf)
                fire_gather(g + K, bf)

        for dg in range(K):
            b = (NB - K + dg) % NB
            wait_gather(b)
            scale(b)
            fire_write(n_chunks - K + dg, b)
        for b in range(NB):
            wait_write(b)

    return lookup


def kernel(x, table):
    R, S = x.shape
    idx = x.astype(jnp.int32)
    out = _make_lookup(R, S, 40, 8, 4)(idx, table)
    return out[:, :, :D_MODEL]
